# baseline (speedup 1.0000x reference)
import functools

import numpy as np
import jax
import jax.numpy as jnp
from jax import lax
from jax.experimental import pallas as pl
from jax.experimental.pallas import tpu as pltpu
from jax.experimental.pallas import tpu_sc as plsc

CAP_FACTOR = 1.25
Z_COEFF = 0.001

NC, NS = 2, 16
NUM_WORKERS = NC * NS
GRP = 32

TOK_BLK = 512
FF_BLK = 512


def _router_body(x_ref, gwt_ref, tril_ref, cslot_ref, counts_ref, psum_ref,
                 logz2_ref, *, capacity, num_slots, tok_per_w, num_experts):
    i = pl.program_id(0)

    @pl.when(i == 0)
    def _init():
        counts_ref[...] = jnp.zeros_like(counts_ref)
        psum_ref[...] = jnp.zeros_like(psum_ref)
        logz2_ref[...] = jnp.zeros_like(logz2_ref)

    x = x_ref[...]
    logits = jnp.dot(x, gwt_ref[...], preferred_element_type=jnp.float32)
    m = jnp.max(logits, axis=-1, keepdims=True)
    ex = jnp.exp(logits - m)
    s = jnp.sum(ex, axis=-1, keepdims=True)
    probs = ex / s
    logz = jnp.log(s) + m
    eids = lax.broadcasted_iota(jnp.int32, logits.shape, 1)
    is_max = logits == m
    expert = jnp.min(jnp.where(is_max, eids, num_experts), axis=-1,
                     keepdims=True)
    onehot = (eids == expert).astype(jnp.float32)
    cum = jnp.dot(tril_ref[...], onehot, preferred_element_type=jnp.float32)
    base = counts_ref[...]
    pos = (jnp.sum((cum + base) * onehot, axis=-1, keepdims=True)
           - 1.0).astype(jnp.int32)
    valid = pos < capacity
    slot = expert * capacity + pos
    tok = i * TOK_BLK + lax.broadcasted_iota(jnp.int32, pos.shape, 0)
    dump = num_slots + tok // tok_per_w
    cslot_ref[...] = jnp.where(valid, slot, dump)[:, 0]
    counts_ref[...] = base + cum[TOK_BLK - 1:TOK_BLK, :]
    psum_ref[...] = psum_ref[...] + jnp.sum(probs, axis=0, keepdims=True)
    logz2_ref[...] = logz2_ref[...] + jnp.sum(logz * logz)


def _router(x, gate_w, capacity, num_slots, tok_per_w):
    n, d = x.shape
    e = gate_w.shape[0]
    nblk = n // TOK_BLK
    tril = jnp.asarray(np.tril(np.ones((TOK_BLK, TOK_BLK), np.float32)))
    body = functools.partial(_router_body, capacity=capacity,
                             num_slots=num_slots, tok_per_w=tok_per_w,
                             num_experts=e)
    return pl.pallas_call(
        body,
        grid=(nblk,),
        in_specs=[
            pl.BlockSpec((TOK_BLK, d), lambda i: (i, 0)),
            pl.BlockSpec((d, e), lambda i: (0, 0)),
            pl.BlockSpec((TOK_BLK, TOK_BLK), lambda i: (0, 0)),
        ],
        out_specs=[
            pl.BlockSpec((TOK_BLK,), lambda i: (i,)),
            pl.BlockSpec((1, e), lambda i: (0, 0)),
            pl.BlockSpec((1, e), lambda i: (0, 0)),
            pl.BlockSpec((1, 1), lambda i: (0, 0)),
        ],
        out_shape=[
            jax.ShapeDtypeStruct((n,), jnp.int32),
            jax.ShapeDtypeStruct((1, e), jnp.float32),
            jax.ShapeDtypeStruct((1, e), jnp.float32),
            jax.ShapeDtypeStruct((1, 1), jnp.float32),
        ],
        compiler_params=pltpu.CompilerParams(
            dimension_semantics=("arbitrary",)),
    )(x, gate_w.T, tril)


def _ffn_body(xe_ref, w1_ref, b1_ref, w2_ref, b2_ref, eo_ref):
    f = pl.program_id(1)
    x = xe_ref[...]
    h = jnp.dot(x, w1_ref[0], preferred_element_type=jnp.float32) + b1_ref[0]
    g = 0.5 * h * (1.0 + lax.erf(h / np.sqrt(2.0)))
    part = jnp.dot(g, w2_ref[0], preferred_element_type=jnp.float32)

    @pl.when(f == 0)
    def _first():
        eo_ref[...] = part + b2_ref[0]

    @pl.when(f != 0)
    def _rest():
        eo_ref[...] = eo_ref[...] + part


def _ffn(xe, w1, b1, w2, b2, capacity):
    e, d, ff = w1.shape
    nf = ff // FF_BLK
    xr = xe.shape[0]
    return pl.pallas_call(
        _ffn_body,
        grid=(e, nf),
        in_specs=[
            pl.BlockSpec((capacity, d), lambda i, j: (i, 0)),
            pl.BlockSpec((1, d, FF_BLK), lambda i, j: (i, 0, j)),
            pl.BlockSpec((1, 1, FF_BLK), lambda i, j: (i, 0, j)),
            pl.BlockSpec((1, FF_BLK, d), lambda i, j: (i, j, 0)),
            pl.BlockSpec((1, 1, d), lambda i, j: (i, 0, 0)),
        ],
        out_specs=pl.BlockSpec((capacity, d), lambda i, j: (i, 0)),
        out_shape=jax.ShapeDtypeStruct((xr, d), jnp.float32),
        compiler_params=pltpu.CompilerParams(
            dimension_semantics=("parallel", "arbitrary")),
    )(xe, w1, b1[:, None, :], w2, b2[:, None, :])


def _sc_mesh():
    return plsc.VectorSubcoreMesh(core_axis_name="c", subcore_axis_name="s",
                                  num_cores=NC, num_subcores=NS)


def _dispatch(x, cslot, xe_rows):
    n, d = x.shape
    tpw = n // NUM_WORKERS
    ng = tpw // GRP

    @functools.partial(
        pl.kernel, mesh=_sc_mesh(),
        out_type=jax.ShapeDtypeStruct((xe_rows, d), jnp.float32),
        scratch_types=[
            pltpu.VMEM((ng, GRP), jnp.int32),
            pltpu.VMEM((GRP, d), jnp.float32),
            pltpu.VMEM((GRP, d), jnp.float32),
            pltpu.SemaphoreType.DMA,
            pltpu.SemaphoreType.DMA,
            pltpu.SemaphoreType.DMA,
            pltpu.SemaphoreType.DMA,
        ],
    )
    def k(x_hbm, cslot_hbm, xe_hbm, idx_v, rows0, rows1, si0, si1, so0, so1):
        wid = lax.axis_index("s") * NC + lax.axis_index("c")
        base = wid * tpw
        pltpu.sync_copy(cslot_hbm.at[wid], idx_v)
        bufs, sin, sout = [rows0, rows1], [si0, si1], [so0, so1]
        in_h, out_h = {}, {}
        in_h[0] = pltpu.async_copy(x_hbm.at[pl.ds(base, GRP), :], bufs[0],
                                   sin[0])
        for g in range(ng):
            b = g % 2
            in_h[g].wait()
            if g + 1 < ng:
                if g - 1 >= 0:
                    out_h[g - 1].wait()
                in_h[g + 1] = pltpu.async_copy(
                    x_hbm.at[pl.ds(base + (g + 1) * GRP, GRP), :],
                    bufs[1 - b], sin[1 - b])
            out_h[g] = pltpu.async_copy(bufs[b], xe_hbm.at[idx_v.at[g]],
                                        sout[b])
        for g in range(max(ng - 2, 0), ng):
            out_h[g].wait()

    return k(x, cslot.reshape(NUM_WORKERS, ng, GRP))


def _combine(x, cslot, eo, num_slots):
    n, d = x.shape
    tpw = n // NUM_WORKERS
    ng = tpw // GRP

    @functools.partial(
        pl.kernel, mesh=_sc_mesh(),
        out_type=jax.ShapeDtypeStruct((n, d), jnp.float32),
        scratch_types=[
            pltpu.VMEM((ng, GRP), jnp.int32),
            pltpu.VMEM((GRP, d), jnp.float32),
            pltpu.VMEM((GRP, d), jnp.float32),
            pltpu.SemaphoreType.DMA,
            pltpu.SemaphoreType.DMA,
            pltpu.SemaphoreType.DMA,
            pltpu.SemaphoreType.DMA,
        ],
    )
    def k(x_hbm, cslot_hbm, eo_hbm, out_hbm, idx_v, rows0, rows1,
          si0, si1, so0, so1):
        wid = lax.axis_index("s") * NC + lax.axis_index("c")
        base = wid * tpw
        pltpu.sync_copy(cslot_hbm.at[wid], idx_v)
        bufs, sin, sout = [rows0, rows1], [si0, si1], [so0, so1]
        in_h, out_h = {}, {}
        in_h[0] = pltpu.async_copy(eo_hbm.at[idx_v.at[0]], bufs[0], sin[0])
        for g in range(ng):
            b = g % 2
            in_h[g].wait()
            if g + 1 < ng:
                if g - 1 >= 0:
                    out_h[g - 1].wait()
                in_h[g + 1] = pltpu.async_copy(eo_hbm.at[idx_v.at[g + 1]],
                                               bufs[1 - b], sin[1 - b])
            tok0 = base + g * GRP
            for h in range(GRP // 16):
                iv = idx_v[g, pl.ds(h * 16, 16)]
                for l in range(16):
                    sidx = iv[l]

                    @pl.when(sidx >= num_slots)
                    def _fix(h=h, l=l, tok0=tok0, buf=bufs[b]):
                        r = h * 16 + l
                        pltpu.sync_copy(x_hbm.at[pl.ds(tok0 + r, 1), :],
                                        buf.at[pl.ds(r, 1), :])
            out_h[g] = pltpu.async_copy(bufs[b],
                                        out_hbm.at[pl.ds(tok0, GRP), :],
                                        sout[b])
        for g in range(max(ng - 2, 0), ng):
            out_h[g].wait()

    return k(x, cslot.reshape(NUM_WORKERS, ng, GRP), eo)


def kernel(hidden, gate_w, w1, b1, w2, b2):
    bx, tx, d = hidden.shape
    n = bx * tx
    e = gate_w.shape[0]
    capacity = max(int(CAP_FACTOR * n / e), 1)
    num_slots = e * capacity
    xe_rows = num_slots + NUM_WORKERS
    tok_per_w = n // NUM_WORKERS
    x = hidden.reshape(n, d)

    cslot, counts, psum, logz2 = _router(x, gate_w, capacity, num_slots,
                                         tok_per_w)
    xe = _dispatch(x, cslot, xe_rows)
    eo = _ffn(xe, w1, b1, w2, b2, capacity)
    out = _combine(x, cslot, eo, num_slots)

    nf = jnp.float32(n)
    z_loss = jnp.float32(Z_COEFF) * (logz2[0, 0] / nf)
    f_i = counts[0] / nf
    p_i = psum[0] / nf
    aux_loss = jnp.float32(e) * jnp.sum(f_i * p_i)
    overflow = jnp.sum(jnp.maximum(counts[0] - jnp.float32(capacity), 0.0))
    overflow_fraction = overflow / nf
    return out.reshape(bx, tx, d), z_loss, aux_loss, overflow_fraction


# speedup vs baseline: 1.1162x; 1.1162x over previous
import functools

import numpy as np
import jax
import jax.numpy as jnp
from jax import lax
from jax.experimental import pallas as pl
from jax.experimental.pallas import tpu as pltpu
from jax.experimental.pallas import tpu_sc as plsc

CAP_FACTOR = 1.25
Z_COEFF = 0.001

NC, NS = 2, 16
NUM_WORKERS = NC * NS
GRP = 32

TOK_BLK = 512
FF_BLK = 1024


def _router_body(x_ref, gwt_ref, tril_ref, cslot_ref, z_ref, aux_ref, ovf_ref,
                 counts_ref, psum_ref, logz2_ref,
                 *, capacity, num_slots, tok_per_w, num_experts):
    i = pl.program_id(0)
    nblk = pl.num_programs(0)

    @pl.when(i == 0)
    def _init():
        counts_ref[...] = jnp.zeros_like(counts_ref)
        psum_ref[...] = jnp.zeros_like(psum_ref)
        logz2_ref[...] = jnp.zeros_like(logz2_ref)

    x = x_ref[...]
    logits = jnp.dot(x, gwt_ref[...], preferred_element_type=jnp.float32)
    m = jnp.max(logits, axis=-1, keepdims=True)
    ex = jnp.exp(logits - m)
    s = jnp.sum(ex, axis=-1, keepdims=True)
    probs = ex / s
    logz = jnp.log(s) + m
    eids = lax.broadcasted_iota(jnp.int32, logits.shape, 1)
    is_max = logits == m
    expert = jnp.min(jnp.where(is_max, eids, num_experts), axis=-1,
                     keepdims=True)
    onehot = (eids == expert).astype(jnp.float32)
    cum = jnp.dot(tril_ref[...], onehot, preferred_element_type=jnp.float32)
    base = counts_ref[...]
    pos = (jnp.sum((cum + base) * onehot, axis=-1, keepdims=True)
           - 1.0).astype(jnp.int32)
    valid = pos < capacity
    slot = expert * capacity + pos
    tok = i * TOK_BLK + lax.broadcasted_iota(jnp.int32, pos.shape, 0)
    dump = num_slots + tok // tok_per_w
    cslot_ref[...] = jnp.where(valid, slot, dump)[:, 0]
    counts_ref[...] = base + cum[TOK_BLK - 1:TOK_BLK, :]
    psum_ref[...] = psum_ref[...] + jnp.sum(probs, axis=0, keepdims=True)
    logz2_ref[...] = logz2_ref[...] + jnp.sum(logz * logz)

    @pl.when(i == nblk - 1)
    def _finalize():
        nf = jnp.float32(tok_per_w) * NUM_WORKERS
        counts = counts_ref[...]
        z_ref[...] = Z_COEFF * logz2_ref[...] / nf
        aux_ref[...] = (num_experts / (nf * nf)) * jnp.sum(
            counts * psum_ref[...], keepdims=True)
        ovf_ref[...] = jnp.sum(
            jnp.maximum(counts - jnp.float32(capacity), 0.0),
            keepdims=True) / nf


def _router(x, gate_w, capacity, num_slots, tok_per_w):
    n, d = x.shape
    e = gate_w.shape[0]
    nblk = n // TOK_BLK
    tril = jnp.asarray(np.tril(np.ones((TOK_BLK, TOK_BLK), np.float32)))
    body = functools.partial(_router_body, capacity=capacity,
                             num_slots=num_slots, tok_per_w=tok_per_w,
                             num_experts=e)
    return pl.pallas_call(
        body,
        grid=(nblk,),
        in_specs=[
            pl.BlockSpec((TOK_BLK, d), lambda i: (i, 0)),
            pl.BlockSpec((d, e), lambda i: (0, 0)),
            pl.BlockSpec((TOK_BLK, TOK_BLK), lambda i: (0, 0)),
        ],
        out_specs=[
            pl.BlockSpec((TOK_BLK,), lambda i: (i,)),
            pl.BlockSpec((1, 1), lambda i: (0, 0)),
            pl.BlockSpec((1, 1), lambda i: (0, 0)),
            pl.BlockSpec((1, 1), lambda i: (0, 0)),
        ],
        out_shape=[
            jax.ShapeDtypeStruct((n,), jnp.int32),
            jax.ShapeDtypeStruct((1, 1), jnp.float32),
            jax.ShapeDtypeStruct((1, 1), jnp.float32),
            jax.ShapeDtypeStruct((1, 1), jnp.float32),
        ],
        scratch_shapes=[
            pltpu.VMEM((1, e), jnp.float32),
            pltpu.VMEM((1, e), jnp.float32),
            pltpu.VMEM((1, 1), jnp.float32),
        ],
        compiler_params=pltpu.CompilerParams(
            dimension_semantics=("arbitrary",)),
    )(x, gate_w.T, tril)


def _ffn_body(xe_ref, w1_ref, b1_ref, w2_ref, b2_ref, eo_ref):
    f = pl.program_id(1)
    x = xe_ref[...]
    h = jnp.dot(x, w1_ref[0], preferred_element_type=jnp.float32) + b1_ref[0]
    g = 0.5 * h * (1.0 + lax.erf(h / np.sqrt(2.0)))
    part = jnp.dot(g, w2_ref[0], preferred_element_type=jnp.float32)

    @pl.when(f == 0)
    def _first():
        eo_ref[...] = part + b2_ref[0]

    @pl.when(f != 0)
    def _rest():
        eo_ref[...] = eo_ref[...] + part


def _ffn(xe, w1, b1, w2, b2, capacity):
    e, d, ff = w1.shape
    nf = ff // FF_BLK
    xr = xe.shape[0]
    return pl.pallas_call(
        _ffn_body,
        grid=(e, nf),
        in_specs=[
            pl.BlockSpec((capacity, d), lambda i, j: (i, 0)),
            pl.BlockSpec((1, d, FF_BLK), lambda i, j: (i, 0, j)),
            pl.BlockSpec((1, 1, FF_BLK), lambda i, j: (i, 0, j)),
            pl.BlockSpec((1, FF_BLK, d), lambda i, j: (i, j, 0)),
            pl.BlockSpec((1, 1, d), lambda i, j: (i, 0, 0)),
        ],
        out_specs=pl.BlockSpec((capacity, d), lambda i, j: (i, 0)),
        out_shape=jax.ShapeDtypeStruct((xr, d), jnp.float32),
        compiler_params=pltpu.CompilerParams(
            dimension_semantics=("parallel", "arbitrary")),
    )(xe, w1, b1[:, None, :], w2, b2[:, None, :])


def _sc_mesh():
    return plsc.VectorSubcoreMesh(core_axis_name="c", subcore_axis_name="s",
                                  num_cores=NC, num_subcores=NS)


def _dispatch(x, cslot, xe_rows):
    n, d = x.shape
    tpw = n // NUM_WORKERS
    ng = tpw // GRP

    @functools.partial(
        pl.kernel, mesh=_sc_mesh(),
        out_type=jax.ShapeDtypeStruct((xe_rows, d), jnp.float32),
        scratch_types=[
            pltpu.VMEM((ng, GRP), jnp.int32),
            pltpu.VMEM((GRP, d), jnp.float32),
            pltpu.VMEM((GRP, d), jnp.float32),
            pltpu.SemaphoreType.DMA,
            pltpu.SemaphoreType.DMA,
            pltpu.SemaphoreType.DMA,
            pltpu.SemaphoreType.DMA,
        ],
    )
    def k(x_hbm, cslot_hbm, xe_hbm, idx_v, rows0, rows1, si0, si1, so0, so1):
        wid = lax.axis_index("s") * NC + lax.axis_index("c")
        base = wid * tpw
        pltpu.sync_copy(cslot_hbm.at[wid], idx_v)
        bufs, sin, sout = [rows0, rows1], [si0, si1], [so0, so1]
        in_h, out_h = {}, {}
        in_h[0] = pltpu.async_copy(x_hbm.at[pl.ds(base, GRP), :], bufs[0],
                                   sin[0])
        for g in range(ng):
            b = g % 2
            in_h[g].wait()
            if g + 1 < ng:
                if g - 1 >= 0:
                    out_h[g - 1].wait()
                in_h[g + 1] = pltpu.async_copy(
                    x_hbm.at[pl.ds(base + (g + 1) * GRP, GRP), :],
                    bufs[1 - b], sin[1 - b])
            out_h[g] = pltpu.async_copy(bufs[b], xe_hbm.at[idx_v.at[g]],
                                        sout[b])
        for g in range(max(ng - 2, 0), ng):
            out_h[g].wait()

    return k(x, cslot.reshape(NUM_WORKERS, ng, GRP))


def _combine(x, cslot, eo, num_slots):
    n, d = x.shape
    tpw = n // NUM_WORKERS
    ng = tpw // GRP

    @functools.partial(
        pl.kernel, mesh=_sc_mesh(),
        out_type=jax.ShapeDtypeStruct((n, d), jnp.float32),
        scratch_types=[
            pltpu.VMEM((ng, GRP), jnp.int32),
            pltpu.VMEM((GRP, d), jnp.float32),
            pltpu.VMEM((GRP, d), jnp.float32),
            pltpu.SemaphoreType.DMA,
            pltpu.SemaphoreType.DMA,
            pltpu.SemaphoreType.DMA,
            pltpu.SemaphoreType.DMA,
        ],
    )
    def k(x_hbm, cslot_hbm, eo_hbm, out_hbm, idx_v, rows0, rows1,
          si0, si1, so0, so1):
        wid = lax.axis_index("s") * NC + lax.axis_index("c")
        base = wid * tpw
        pltpu.sync_copy(cslot_hbm.at[wid], idx_v)
        bufs, sin, sout = [rows0, rows1], [si0, si1], [so0, so1]
        in_h, out_h = {}, {}
        in_h[0] = pltpu.async_copy(eo_hbm.at[idx_v.at[0]], bufs[0], sin[0])
        for g in range(ng):
            b = g % 2
            in_h[g].wait()
            if g + 1 < ng:
                if g - 1 >= 0:
                    out_h[g - 1].wait()
                in_h[g + 1] = pltpu.async_copy(eo_hbm.at[idx_v.at[g + 1]],
                                               bufs[1 - b], sin[1 - b])
            tok0 = base + g * GRP
            for h in range(GRP // 16):
                iv = idx_v[g, pl.ds(h * 16, 16)]
                for l in range(16):
                    sidx = iv[l]

                    @pl.when(sidx >= num_slots)
                    def _fix(h=h, l=l, tok0=tok0, buf=bufs[b]):
                        r = h * 16 + l
                        pltpu.sync_copy(x_hbm.at[pl.ds(tok0 + r, 1), :],
                                        buf.at[pl.ds(r, 1), :])
            out_h[g] = pltpu.async_copy(bufs[b],
                                        out_hbm.at[pl.ds(tok0, GRP), :],
                                        sout[b])
        for g in range(max(ng - 2, 0), ng):
            out_h[g].wait()

    return k(x, cslot.reshape(NUM_WORKERS, ng, GRP), eo)


def kernel(hidden, gate_w, w1, b1, w2, b2):
    bx, tx, d = hidden.shape
    n = bx * tx
    e = gate_w.shape[0]
    capacity = max(int(CAP_FACTOR * n / e), 1)
    num_slots = e * capacity
    xe_rows = num_slots + NUM_WORKERS
    tok_per_w = n // NUM_WORKERS
    x = hidden.reshape(n, d)

    cslot, z_loss, aux_loss, ovf = _router(x, gate_w, capacity, num_slots,
                                           tok_per_w)
    xe = _dispatch(x, cslot, xe_rows)
    eo = _ffn(xe, w1, b1, w2, b2, capacity)
    out = _combine(x, cslot, eo, num_slots)
    return (out.reshape(bx, tx, d), z_loss[0, 0], aux_loss[0, 0], ovf[0, 0])
